# baseline (speedup 1.0000x reference)
import functools

import jax
import jax.numpy as jnp
import numpy as np
from jax import lax
from jax.experimental import pallas as pl
from jax.experimental.pallas import tpu as pltpu
from jax.experimental.pallas import tpu_sc as plsc

N_POINTS = 262144
N_LEVELS = 12
TABLE_SIZE = 2 ** 19
MASK = TABLE_SIZE - 1
BASE_RES = 16
LEVEL_SCALE = 1.38
IN_DIM = 2 * N_LEVELS
WIDTH = 64

RES = [float(np.floor(BASE_RES * LEVEL_SCALE ** l)) for l in range(N_LEVELS)]

P2 = int(np.uint32(2654435761).astype(np.int32))
P3 = int(np.uint32(805459861).astype(np.int32))

NC = 2
NS = 16
NW = NC * NS
LANES = 16

CORNERS = [(cx, cy, cz) for cx in (0, 1) for cy in (0, 1) for cz in (0, 1)]


def make_sc_encode(n_points, c=128, interpret=False):
    C = c
    G = C // LANES
    ROWS = C * 8 * N_LEVELS
    NELEM = 2 * ROWS
    DMA_ROWS = 128
    NDMA = NELEM // DMA_ROWS
    PER_W = n_points // NW
    NCHUNK = PER_W // C

    def _sc_encode_body(xn_hbm, tbl_hbm, pe_hbm, xv, idxv, rowsv, pev, sem):
        wid = lax.axis_index("s") * NC + lax.axis_index("c")
        wbase = wid * PER_W

        lane = lax.iota(jnp.int32, LANES)

        def load_xyz(g):
            base3 = g * (LANES * 3)
            x0 = plsc.load_gather(xv, [base3 + lane * 3])
            x1 = plsc.load_gather(xv, [base3 + lane * 3 + 1])
            x2 = plsc.load_gather(xv, [base3 + lane * 3 + 2])
            return x0, x1, x2

        def chunk_body(ci, carry):
            base = wbase + ci * C
            pltpu.sync_copy(xn_hbm.at[pl.ds(base * 3, C * 3)], xv)

            def pass_a(g, c2):
                x0, x1, x2 = load_xyz(g)
                for l in range(N_LEVELS):
                    pos0 = x0 * RES[l]
                    pos1 = x1 * RES[l]
                    pos2 = x2 * RES[l]
                    a0 = pos0.astype(jnp.int32)
                    b0 = pos1.astype(jnp.int32) * P2
                    c0 = pos2.astype(jnp.int32) * P3
                    a1 = a0 + 1
                    b1 = b0 + P2
                    c1 = c0 + P3
                    loff = l * TABLE_SIZE
                    bc = [[b0 ^ c0, b0 ^ c1], [b1 ^ c0, b1 ^ c1]]
                    for k, (cx, cy, cz) in enumerate(CORNERS):
                        h = (a1 if cx else a0) ^ bc[cy][cz]
                        e0 = (((h & MASK) + loff) << 1)
                        sbase = (l * 8 + k) * 2 * C + g * LANES
                        idxv[pl.ds(sbase, LANES)] = e0
                        idxv[pl.ds(sbase + C, LANES)] = e0 + 1
                return c2

            lax.fori_loop(0, G, pass_a, 0)

            def fire(j, c2):
                pltpu.async_copy(
                    tbl_hbm.at[idxv.at[pl.ds(j * DMA_ROWS, DMA_ROWS)]],
                    rowsv.at[pl.ds(j * DMA_ROWS, DMA_ROWS)],
                    sem,
                )
                return c2

            lax.fori_loop(0, NDMA, fire, 0)

            def drain(j, c2):
                pltpu.make_async_copy(
                    tbl_hbm.at[idxv.at[pl.ds(j * DMA_ROWS, DMA_ROWS)]],
                    rowsv.at[pl.ds(j * DMA_ROWS, DMA_ROWS)],
                    sem,
                ).wait()
                return c2

            lax.fori_loop(0, NDMA, drain, 0)

            def pass_b(g, c2):
                x0, x1, x2 = load_xyz(g)
                for l in range(N_LEVELS):
                    pos0 = x0 * RES[l]
                    pos1 = x1 * RES[l]
                    pos2 = x2 * RES[l]
                    t0 = pos0 - pos0.astype(jnp.int32).astype(jnp.float32)
                    t1 = pos1 - pos1.astype(jnp.int32).astype(jnp.float32)
                    t2 = pos2 - pos2.astype(jnp.int32).astype(jnp.float32)
                    u0 = 1.0 - t0
                    u1 = 1.0 - t1
                    u2 = 1.0 - t2
                    wyz = [[u1 * u2, u1 * t2], [t1 * u2, t1 * t2]]
                    interp0 = jnp.zeros((LANES,), jnp.float32)
                    interp1 = jnp.zeros((LANES,), jnp.float32)
                    for k, (cx, cy, cz) in enumerate(CORNERS):
                        ridx = (l * 8 + k) * 2 * C + g * LANES + lane
                        f0 = plsc.load_gather(rowsv, [ridx])
                        f1 = plsc.load_gather(rowsv, [ridx + C])
                        w = (t0 if cx else u0) * wyz[cy][cz]
                        interp0 = interp0 + f0 * w
                        interp1 = interp1 + f1 * w
                    pev[2 * l, pl.ds(g * LANES, LANES)] = interp0
                    pev[2 * l + 1, pl.ds(g * LANES, LANES)] = interp1
                return c2

            lax.fori_loop(0, G, pass_b, 0)

            pltpu.sync_copy(pev, pe_hbm.at[:, pl.ds(base, C)])
            return carry

        lax.fori_loop(0, NCHUNK, chunk_body, 0)

    return functools.partial(
        pl.kernel,
        out_type=jax.ShapeDtypeStruct((IN_DIM, n_points), jnp.float32),
        mesh=plsc.VectorSubcoreMesh(
            core_axis_name="c", subcore_axis_name="s", num_cores=NC, num_subcores=NS
        ),
        compiler_params=pltpu.CompilerParams(
            needs_layout_passes=False, use_tc_tiling_on_sc=False
        ),
        scratch_types=[
            pltpu.VMEM((C * 3,), jnp.float32),
            pltpu.VMEM((NELEM,), jnp.int32),
            pltpu.VMEM((NELEM,), jnp.float32),
            pltpu.VMEM((IN_DIM, C), jnp.float32),
            pltpu.SemaphoreType.DMA,
        ],
        interpret=interpret,
    )(_sc_encode_body)


_sc_encode = make_sc_encode(N_POINTS)


BN = 2048


def _mlp_body(pe_ref, w0t_ref, b0_ref, w1t_ref, b1_ref, w2r_ref, b2r_ref, o_ref):
    pe = pe_ref[...]
    h = lax.dot(w0t_ref[...], pe, preferred_element_type=jnp.float32)
    h = jnp.maximum(h + b0_ref[...], 0.0)
    h = lax.dot(w1t_ref[...], h, preferred_element_type=jnp.float32)
    h = jnp.maximum(h + b1_ref[...], 0.0)
    z = lax.dot(w2r_ref[...], h, preferred_element_type=jnp.float32) + b2r_ref[...]
    o_ref[...] = jnp.maximum(z, 0.0) + jnp.log(1.0 + jnp.exp(-jnp.abs(z)))


_mlp = pl.pallas_call(
    _mlp_body,
    grid=(N_POINTS // BN,),
    in_specs=[
        pl.BlockSpec((IN_DIM, BN), lambda i: (0, i)),
        pl.BlockSpec((WIDTH, IN_DIM), lambda i: (0, 0)),
        pl.BlockSpec((WIDTH, 1), lambda i: (0, 0)),
        pl.BlockSpec((WIDTH, WIDTH), lambda i: (0, 0)),
        pl.BlockSpec((WIDTH, 1), lambda i: (0, 0)),
        pl.BlockSpec((1, WIDTH), lambda i: (0, 0)),
        pl.BlockSpec((1, 1), lambda i: (0, 0)),
    ],
    out_specs=pl.BlockSpec((1, BN), lambda i: (0, i)),
    out_shape=jax.ShapeDtypeStruct((1, N_POINTS), jnp.float32),
)


def kernel(x, bounding_box, tables, W0, b0, W1, b1, W2, b2):
    xn = (x - bounding_box[0]) / (bounding_box[1] - bounding_box[0])
    pe = _sc_encode(xn.reshape(-1), tables.reshape(-1))
    dens = _mlp(
        pe,
        W0.T,
        b0.reshape(WIDTH, 1),
        W1.T,
        b1.reshape(WIDTH, 1),
        W2[:, 0:1].T,
        b2[0:1].reshape(1, 1),
    )
    return dens.reshape(N_POINTS)


# speedup vs baseline: 1.0263x; 1.0263x over previous
import functools

import jax
import jax.numpy as jnp
import numpy as np
from jax import lax
from jax.experimental import pallas as pl
from jax.experimental.pallas import tpu as pltpu
from jax.experimental.pallas import tpu_sc as plsc

N_POINTS = 262144
N_LEVELS = 12
TABLE_SIZE = 2 ** 19
MASK = TABLE_SIZE - 1
BASE_RES = 16
LEVEL_SCALE = 1.38
IN_DIM = 2 * N_LEVELS
WIDTH = 64

RES = [float(np.floor(BASE_RES * LEVEL_SCALE ** l)) for l in range(N_LEVELS)]

P2 = int(np.uint32(2654435761).astype(np.int32))
P3 = int(np.uint32(805459861).astype(np.int32))

NC = 2
NS = 16
NW = NC * NS
LANES = 16

NSUPER = N_LEVELS * TABLE_SIZE * 2 // 8

CORNERS = [(cx, cy, cz) for cx in (0, 1) for cy in (0, 1) for cz in (0, 1)]


def make_sc_encode(n_points, c=128, interpret=False):
    C = c
    G = C // LANES
    ROWS = C * 8 * N_LEVELS
    DMA_ROWS = 128
    NDMA = ROWS // DMA_ROWS
    PER_W = n_points // NW
    NCHUNK = PER_W // C
    NTCOL = n_points // 128

    def corner_parts(x0, x1, x2, l):
        pos0 = x0 * RES[l]
        pos1 = x1 * RES[l]
        pos2 = x2 * RES[l]
        a0 = pos0.astype(jnp.int32)
        b0 = pos1.astype(jnp.int32) * P2
        c0 = pos2.astype(jnp.int32) * P3
        parts = (
            [a0, a0 + 1],
            [b0 ^ c0, b0 ^ (c0 + P3), (b0 + P2) ^ c0, (b0 + P2) ^ (c0 + P3)],
        )
        return (pos0, pos1, pos2), parts

    def _sc_encode_body(xn_hbm, tbl_hbm, pe_hbm, xv, idxv, rowsv, pev, sem):
        wid = lax.axis_index("s") * NC + lax.axis_index("c")
        wbase = wid * PER_W

        lane = lax.iota(jnp.int32, LANES)

        def chunk_body(ci, carry):
            base = wbase + ci * C
            for d in range(3):
                pltpu.sync_copy(
                    xn_hbm.at[pl.ds(d * n_points + base, C)],
                    xv.at[pl.ds(d * C, C)],
                )

            def load_xyz(g):
                x0 = xv[pl.ds(g * LANES, LANES)]
                x1 = xv[pl.ds(C + g * LANES, LANES)]
                x2 = xv[pl.ds(2 * C + g * LANES, LANES)]
                return x0, x1, x2

            def pass_a(g, c2):
                x0, x1, x2 = load_xyz(g)
                for l in range(N_LEVELS):
                    _, (av, bcv) = corner_parts(x0, x1, x2, l)
                    loff = l * TABLE_SIZE
                    for k, (cx, cy, cz) in enumerate(CORNERS):
                        h = av[cx] ^ bcv[cy * 2 + cz]
                        m = (h & MASK) + loff
                        idxv[pl.ds((l * 8 + k) * C + g * LANES, LANES)] = (
                            m >> 2
                        )
                return c2

            lax.fori_loop(0, G, pass_a, 0)

            def fire(j, c2):
                pltpu.async_copy(
                    tbl_hbm.at[idxv.at[pl.ds(j * DMA_ROWS, DMA_ROWS)]],
                    rowsv.at[pl.ds(j * DMA_ROWS, DMA_ROWS)],
                    sem,
                )
                return c2

            lax.fori_loop(0, NDMA, fire, 0)

            def drain(j, c2):
                pltpu.make_async_copy(
                    tbl_hbm.at[idxv.at[pl.ds(j * DMA_ROWS, DMA_ROWS)]],
                    rowsv.at[pl.ds(j * DMA_ROWS, DMA_ROWS)],
                    sem,
                ).wait()
                return c2

            lax.fori_loop(0, NDMA, drain, 0)

            def pass_b(g, c2):
                x0, x1, x2 = load_xyz(g)
                for l in range(N_LEVELS):
                    (pos0, pos1, pos2), (av, bcv) = corner_parts(x0, x1, x2, l)
                    t0 = pos0 - pos0.astype(jnp.int32).astype(jnp.float32)
                    t1 = pos1 - pos1.astype(jnp.int32).astype(jnp.float32)
                    t2 = pos2 - pos2.astype(jnp.int32).astype(jnp.float32)
                    u0 = 1.0 - t0
                    u1 = 1.0 - t1
                    u2 = 1.0 - t2
                    wyz = [u1 * u2, u1 * t2, t1 * u2, t1 * t2]
                    interp0 = jnp.zeros((LANES,), jnp.float32)
                    interp1 = jnp.zeros((LANES,), jnp.float32)
                    for k, (cx, cy, cz) in enumerate(CORNERS):
                        h = av[cx] ^ bcv[cy * 2 + cz]
                        col0 = (h & 3) << 1
                        ridx = (l * 8 + k) * C + g * LANES + lane
                        f0 = plsc.load_gather(rowsv, [ridx, col0])
                        f1 = plsc.load_gather(rowsv, [ridx, col0 + 1])
                        w = (t0 if cx else u0) * wyz[cy * 2 + cz]
                        interp0 = interp0 + f0 * w
                        interp1 = interp1 + f1 * w
                    pev[pl.ds(2 * l * C + g * LANES, LANES)] = interp0
                    pev[pl.ds((2 * l + 1) * C + g * LANES, LANES)] = interp1
                return c2

            lax.fori_loop(0, G, pass_b, 0)

            tcol = base // 128
            for t in range(3):
                pltpu.sync_copy(
                    pev.at[pl.ds(t * 8 * C, 8 * C)],
                    pe_hbm.at[pl.ds((t * NTCOL + tcol) * 1024, 1024)],
                )
            return carry

        lax.fori_loop(0, NCHUNK, chunk_body, 0)

    return functools.partial(
        pl.kernel,
        out_type=jax.ShapeDtypeStruct((IN_DIM * n_points,), jnp.float32),
        mesh=plsc.VectorSubcoreMesh(
            core_axis_name="c", subcore_axis_name="s", num_cores=NC, num_subcores=NS
        ),
        compiler_params=pltpu.CompilerParams(
            needs_layout_passes=False, use_tc_tiling_on_sc=False
        ),
        scratch_types=[
            pltpu.VMEM((C * 3,), jnp.float32),
            pltpu.VMEM((ROWS,), jnp.int32),
            pltpu.VMEM((ROWS, 8), jnp.float32),
            pltpu.VMEM((IN_DIM * C,), jnp.float32),
            pltpu.SemaphoreType.DMA,
        ],
        interpret=interpret,
    )(_sc_encode_body)


_sc_encode = make_sc_encode(N_POINTS)


BN = 2048
BNC = BN // 128


def _mlp_body(pe_ref, w0t_ref, b0_ref, w1t_ref, b1_ref, w2r_ref, b2r_ref, o_ref):
    w0t = w0t_ref[...]
    b0 = b0_ref[...]
    w1t = w1t_ref[...]
    b1 = b1_ref[...]
    w2r = w2r_ref[...]
    b2 = b2r_ref[...]
    for cb in range(BNC):
        s = None
        for t in range(3):
            d = lax.dot(
                w0t[:, t * 8 : (t + 1) * 8],
                pe_ref[t, cb],
                preferred_element_type=jnp.float32,
            )
            s = d if s is None else s + d
        h = jnp.maximum(s + b0, 0.0)
        h = jnp.maximum(
            lax.dot(w1t, h, preferred_element_type=jnp.float32) + b1, 0.0
        )
        z = lax.dot(w2r, h, preferred_element_type=jnp.float32) + b2
        o_ref[:, pl.ds(cb * 128, 128)] = jnp.maximum(z, 0.0) + jnp.log(
            1.0 + jnp.exp(-jnp.abs(z))
        )


_mlp = pl.pallas_call(
    _mlp_body,
    grid=(N_POINTS // BN,),
    in_specs=[
        pl.BlockSpec((3, BNC, 8, 128), lambda i: (0, i, 0, 0)),
        pl.BlockSpec((WIDTH, IN_DIM), lambda i: (0, 0)),
        pl.BlockSpec((WIDTH, 1), lambda i: (0, 0)),
        pl.BlockSpec((WIDTH, WIDTH), lambda i: (0, 0)),
        pl.BlockSpec((WIDTH, 1), lambda i: (0, 0)),
        pl.BlockSpec((1, WIDTH), lambda i: (0, 0)),
        pl.BlockSpec((1, 1), lambda i: (0, 0)),
    ],
    out_specs=pl.BlockSpec((1, BN), lambda i: (0, i)),
    out_shape=jax.ShapeDtypeStruct((1, N_POINTS), jnp.float32),
)


def kernel(x, bounding_box, tables, W0, b0, W1, b1, W2, b2):
    xn = ((x - bounding_box[0]) / (bounding_box[1] - bounding_box[0])).T
    pe_flat = _sc_encode(xn.reshape(-1), tables.reshape(NSUPER, 8))
    pe4 = pe_flat.reshape(3, N_POINTS // 128, 8, 128)
    dens = _mlp(
        pe4,
        W0.T,
        b0.reshape(WIDTH, 1),
        W1.T,
        b1.reshape(WIDTH, 1),
        W2[:, 0:1].T,
        b2[0:1].reshape(1, 1),
    )
    return dens.reshape(N_POINTS)


# speedup vs baseline: 3.7240x; 3.6284x over previous
import functools

import jax
import jax.numpy as jnp
import numpy as np
from jax import lax
from jax.experimental import pallas as pl
from jax.experimental.pallas import tpu as pltpu
from jax.experimental.pallas import tpu_sc as plsc

N_POINTS = 262144
N_LEVELS = 12
TABLE_SIZE = 2 ** 19
MASK = TABLE_SIZE - 1
BASE_RES = 16
LEVEL_SCALE = 1.38
IN_DIM = 2 * N_LEVELS
WIDTH = 64

RES = [float(np.floor(BASE_RES * LEVEL_SCALE ** l)) for l in range(N_LEVELS)]

P2 = int(np.uint32(2654435761).astype(np.int32))
P3 = int(np.uint32(805459861).astype(np.int32))

NC = 2
NS = 16
NW = NC * NS
LANES = 16

LVL_STRIDE = TABLE_SIZE * 2

CORNERS = [(cx, cy, cz) for cx in (0, 1) for cy in (0, 1) for cz in (0, 1)]


def make_sc_encode(n_points, c=128, interpret=False):
    C = c
    G = C // LANES
    ROWS = C * 8 * N_LEVELS
    NELEM = 2 * ROWS
    DMA_ROWS = 128
    NDMA = NELEM // DMA_ROWS
    PER_W = n_points // NW
    NCHUNK = PER_W // C
    NTCOL = n_points // 128

    def corner_parts(x0, x1, x2, l):
        pos0 = x0 * RES[l]
        pos1 = x1 * RES[l]
        pos2 = x2 * RES[l]
        a0 = pos0.astype(jnp.int32)
        b0 = pos1.astype(jnp.int32) * P2
        c0 = pos2.astype(jnp.int32) * P3
        parts = (
            [a0, a0 + 1],
            [b0 ^ c0, b0 ^ (c0 + P3), (b0 + P2) ^ c0, (b0 + P2) ^ (c0 + P3)],
        )
        return (pos0, pos1, pos2), parts

    def _sc_encode_body(xn_hbm, tbl_hbm, pe_hbm, xv, idxv, rowsv, pev, sem):
        wid = lax.axis_index("s") * NC + lax.axis_index("c")
        wbase = wid * PER_W

        lane = lax.iota(jnp.int32, LANES)

        def chunk_body(ci, carry):
            base = wbase + ci * C
            for d in range(3):
                pltpu.sync_copy(
                    xn_hbm.at[pl.ds(d * n_points + base, C)],
                    xv.at[pl.ds(d * C, C)],
                )

            def load_xyz(g):
                x0 = xv[pl.ds(g * LANES, LANES)]
                x1 = xv[pl.ds(C + g * LANES, LANES)]
                x2 = xv[pl.ds(2 * C + g * LANES, LANES)]
                return x0, x1, x2

            def pass_a(g, c2):
                x0, x1, x2 = load_xyz(g)
                for l in range(N_LEVELS):
                    _, (av, bcv) = corner_parts(x0, x1, x2, l)
                    loff = l * LVL_STRIDE
                    for k, (cx, cy, cz) in enumerate(CORNERS):
                        h = av[cx] ^ bcv[cy * 2 + cz]
                        m = h & MASK
                        o0 = loff + ((m & ~127) << 1) + (m & 127)
                        sbase = (l * 8 + k) * 2 * C + g * LANES
                        idxv[pl.ds(sbase, LANES)] = o0
                        idxv[pl.ds(sbase + C, LANES)] = o0 + 128
                return c2

            lax.fori_loop(0, G, pass_a, 0)

            def fire(j, c2):
                pltpu.async_copy(
                    tbl_hbm.at[idxv.at[pl.ds(j * DMA_ROWS, DMA_ROWS)]],
                    rowsv.at[pl.ds(j * DMA_ROWS, DMA_ROWS)],
                    sem,
                )
                return c2

            lax.fori_loop(0, NDMA, fire, 0)

            def drain(j, c2):
                pltpu.make_async_copy(
                    tbl_hbm.at[idxv.at[pl.ds(j * DMA_ROWS, DMA_ROWS)]],
                    rowsv.at[pl.ds(j * DMA_ROWS, DMA_ROWS)],
                    sem,
                ).wait()
                return c2

            lax.fori_loop(0, NDMA, drain, 0)

            def pass_b(g, c2):
                x0, x1, x2 = load_xyz(g)
                for l in range(N_LEVELS):
                    (pos0, pos1, pos2), (av, bcv) = corner_parts(x0, x1, x2, l)
                    t0 = pos0 - pos0.astype(jnp.int32).astype(jnp.float32)
                    t1 = pos1 - pos1.astype(jnp.int32).astype(jnp.float32)
                    t2 = pos2 - pos2.astype(jnp.int32).astype(jnp.float32)
                    u0 = 1.0 - t0
                    u1 = 1.0 - t1
                    u2 = 1.0 - t2
                    wyz = [u1 * u2, u1 * t2, t1 * u2, t1 * t2]
                    interp0 = jnp.zeros((LANES,), jnp.float32)
                    interp1 = jnp.zeros((LANES,), jnp.float32)
                    for k, (cx, cy, cz) in enumerate(CORNERS):
                        ridx = (l * 8 + k) * 2 * C + g * LANES + lane
                        f0 = plsc.load_gather(rowsv, [ridx])
                        f1 = plsc.load_gather(rowsv, [ridx + C])
                        w = (t0 if cx else u0) * wyz[cy * 2 + cz]
                        interp0 = interp0 + f0 * w
                        interp1 = interp1 + f1 * w
                    pev[pl.ds(2 * l * C + g * LANES, LANES)] = interp0
                    pev[pl.ds((2 * l + 1) * C + g * LANES, LANES)] = interp1
                return c2

            lax.fori_loop(0, G, pass_b, 0)

            tcol = base // 128
            for t in range(3):
                pltpu.sync_copy(
                    pev.at[pl.ds(t * 8 * C, 8 * C)],
                    pe_hbm.at[pl.ds((t * NTCOL + tcol) * 1024, 1024)],
                )
            return carry

        lax.fori_loop(0, NCHUNK, chunk_body, 0)

    return functools.partial(
        pl.kernel,
        out_type=jax.ShapeDtypeStruct((IN_DIM * n_points,), jnp.float32),
        mesh=plsc.VectorSubcoreMesh(
            core_axis_name="c", subcore_axis_name="s", num_cores=NC, num_subcores=NS
        ),
        compiler_params=pltpu.CompilerParams(
            needs_layout_passes=False, use_tc_tiling_on_sc=False
        ),
        scratch_types=[
            pltpu.VMEM((C * 3,), jnp.float32),
            pltpu.VMEM((NELEM,), jnp.int32),
            pltpu.VMEM((NELEM,), jnp.float32),
            pltpu.VMEM((IN_DIM * C,), jnp.float32),
            pltpu.SemaphoreType.DMA,
        ],
        interpret=interpret,
    )(_sc_encode_body)


_sc_encode = make_sc_encode(N_POINTS)


BN = 2048
BNC = BN // 128


def _mlp_body(pe_ref, w0t_ref, b0_ref, w1t_ref, b1_ref, w2r_ref, b2r_ref, o_ref):
    w0t = w0t_ref[...]
    b0 = b0_ref[...]
    w1t = w1t_ref[...]
    b1 = b1_ref[...]
    w2r = w2r_ref[...]
    b2 = b2r_ref[...]
    for cb in range(BNC):
        s = None
        for t in range(3):
            d = lax.dot(
                w0t[:, t * 8 : (t + 1) * 8],
                pe_ref[t, cb],
                preferred_element_type=jnp.float32,
            )
            s = d if s is None else s + d
        h = jnp.maximum(s + b0, 0.0)
        h = jnp.maximum(
            lax.dot(w1t, h, preferred_element_type=jnp.float32) + b1, 0.0
        )
        z = lax.dot(w2r, h, preferred_element_type=jnp.float32) + b2
        o_ref[:, pl.ds(cb * 128, 128)] = jnp.maximum(z, 0.0) + jnp.log(
            1.0 + jnp.exp(-jnp.abs(z))
        )


_mlp = pl.pallas_call(
    _mlp_body,
    grid=(N_POINTS // BN,),
    in_specs=[
        pl.BlockSpec((3, BNC, 8, 128), lambda i: (0, i, 0, 0)),
        pl.BlockSpec((WIDTH, IN_DIM), lambda i: (0, 0)),
        pl.BlockSpec((WIDTH, 1), lambda i: (0, 0)),
        pl.BlockSpec((WIDTH, WIDTH), lambda i: (0, 0)),
        pl.BlockSpec((WIDTH, 1), lambda i: (0, 0)),
        pl.BlockSpec((1, WIDTH), lambda i: (0, 0)),
        pl.BlockSpec((1, 1), lambda i: (0, 0)),
    ],
    out_specs=pl.BlockSpec((1, BN), lambda i: (0, i)),
    out_shape=jax.ShapeDtypeStruct((1, N_POINTS), jnp.float32),
)


def kernel(x, bounding_box, tables, W0, b0, W1, b1, W2, b2):
    xn = ((x - bounding_box[0]) / (bounding_box[1] - bounding_box[0])).T
    tbl_lin = (
        tables.reshape(N_LEVELS, TABLE_SIZE // 128, 128, 2)
        .transpose(0, 1, 3, 2)
        .reshape(-1)
    )
    pe_flat = _sc_encode(xn.reshape(-1), tbl_lin)
    pe4 = pe_flat.reshape(3, N_POINTS // 128, 8, 128)
    dens = _mlp(
        pe4,
        W0.T,
        b0.reshape(WIDTH, 1),
        W1.T,
        b1.reshape(WIDTH, 1),
        W2[:, 0:1].T,
        b2[0:1].reshape(1, 1),
    )
    return dens.reshape(N_POINTS)


# speedup vs baseline: 5.0804x; 1.3642x over previous
import functools

import jax
import jax.numpy as jnp
import numpy as np
from jax import lax
from jax.experimental import pallas as pl
from jax.experimental.pallas import tpu as pltpu
from jax.experimental.pallas import tpu_sc as plsc

N_POINTS = 262144
N_LEVELS = 12
TABLE_SIZE = 2 ** 19
MASK = TABLE_SIZE - 1
BASE_RES = 16
LEVEL_SCALE = 1.38
IN_DIM = 2 * N_LEVELS
WIDTH = 64

RES = [float(np.floor(BASE_RES * LEVEL_SCALE ** l)) for l in range(N_LEVELS)]

P2 = int(np.uint32(2654435761).astype(np.int32))
P3 = int(np.uint32(805459861).astype(np.int32))

NC = 2
NS = 16
NW = NC * NS
LANES = 16

NTBL = N_LEVELS * TABLE_SIZE * 2
NSUPER = NTBL // 8

CORNERS = [(cx, cy, cz) for cx in (0, 1) for cy in (0, 1) for cz in (0, 1)]


ICHUNK = 16384


def _sc_interleave_body(src_hbm, dst_hbm, inv, outv):
    wid = lax.axis_index("s") * NC + lax.axis_index("c")
    per_w = NTBL // NW
    wbase = wid * per_w
    nchunk = per_w // ICHUNK
    lane = lax.iota(jnp.int32, LANES)
    perm = (lane >> 1) + (lane & 1) * 128

    def chunk(ci, carry):
        base = wbase + ci * ICHUNK
        pltpu.sync_copy(src_hbm.at[pl.ds(base, ICHUNK)], inv)

        def block(b, c2):
            boff = b * 256

            def group(gi, c3):
                vals = plsc.load_gather(inv, [boff + gi * 8 + perm])
                outv[pl.ds(boff + gi * LANES, LANES)] = vals
                return c3

            lax.fori_loop(0, 16, group, 0)
            return c2

        lax.fori_loop(0, ICHUNK // 256, block, 0)
        pltpu.sync_copy(outv, dst_hbm.at[pl.ds(base, ICHUNK)])
        return carry

    lax.fori_loop(0, nchunk, chunk, 0)


_sc_interleave = functools.partial(
    pl.kernel,
    out_type=jax.ShapeDtypeStruct((NTBL,), jnp.float32),
    mesh=plsc.VectorSubcoreMesh(
        core_axis_name="c", subcore_axis_name="s", num_cores=NC, num_subcores=NS
    ),
    compiler_params=pltpu.CompilerParams(
        needs_layout_passes=False, use_tc_tiling_on_sc=False
    ),
    scratch_types=[
        pltpu.VMEM((ICHUNK,), jnp.float32),
        pltpu.VMEM((ICHUNK,), jnp.float32),
    ],
)(_sc_interleave_body)


def make_sc_encode(n_points, c=128, interpret=False):
    C = c
    G = C // LANES
    ROWS = C * 8 * N_LEVELS
    DMA_ROWS = 128
    NDMA = ROWS // DMA_ROWS
    PER_W = n_points // NW
    NCHUNK = PER_W // C
    NTCOL = n_points // 128

    def corner_parts(x0, x1, x2, l):
        pos0 = x0 * RES[l]
        pos1 = x1 * RES[l]
        pos2 = x2 * RES[l]
        a0 = pos0.astype(jnp.int32)
        b0 = pos1.astype(jnp.int32) * P2
        c0 = pos2.astype(jnp.int32) * P3
        parts = (
            [a0, a0 + 1],
            [b0 ^ c0, b0 ^ (c0 + P3), (b0 + P2) ^ c0, (b0 + P2) ^ (c0 + P3)],
        )
        return (pos0, pos1, pos2), parts

    def _sc_encode_body(xn_hbm, tbl_hbm, pe_hbm, xv, idxv, rowsv, pev, sem):
        wid = lax.axis_index("s") * NC + lax.axis_index("c")
        wbase = wid * PER_W

        lane = lax.iota(jnp.int32, LANES)

        def chunk_body(ci, carry):
            base = wbase + ci * C
            for d in range(3):
                pltpu.sync_copy(
                    xn_hbm.at[pl.ds(d * n_points + base, C)],
                    xv.at[pl.ds(d * C, C)],
                )

            def load_xyz(g):
                x0 = xv[pl.ds(g * LANES, LANES)]
                x1 = xv[pl.ds(C + g * LANES, LANES)]
                x2 = xv[pl.ds(2 * C + g * LANES, LANES)]
                return x0, x1, x2

            def pass_a(g, c2):
                x0, x1, x2 = load_xyz(g)
                for l in range(N_LEVELS):
                    _, (av, bcv) = corner_parts(x0, x1, x2, l)
                    loff = l * (TABLE_SIZE // 4)
                    for k, (cx, cy, cz) in enumerate(CORNERS):
                        h = av[cx] ^ bcv[cy * 2 + cz]
                        idxv[pl.ds((l * 8 + k) * C + g * LANES, LANES)] = (
                            loff + ((h & MASK) >> 2)
                        )
                return c2

            lax.fori_loop(0, G, pass_a, 0)

            def fire(j, c2):
                pltpu.async_copy(
                    tbl_hbm.at[idxv.at[pl.ds(j * DMA_ROWS, DMA_ROWS)]],
                    rowsv.at[pl.ds(j * DMA_ROWS, DMA_ROWS)],
                    sem,
                )
                return c2

            lax.fori_loop(0, NDMA, fire, 0)

            def drain(j, c2):
                pltpu.make_async_copy(
                    tbl_hbm.at[idxv.at[pl.ds(j * DMA_ROWS, DMA_ROWS)]],
                    rowsv.at[pl.ds(j * DMA_ROWS, DMA_ROWS)],
                    sem,
                ).wait()
                return c2

            lax.fori_loop(0, NDMA, drain, 0)

            def pass_b(g, c2):
                x0, x1, x2 = load_xyz(g)
                for l in range(N_LEVELS):
                    (pos0, pos1, pos2), (av, bcv) = corner_parts(x0, x1, x2, l)
                    t0 = pos0 - pos0.astype(jnp.int32).astype(jnp.float32)
                    t1 = pos1 - pos1.astype(jnp.int32).astype(jnp.float32)
                    t2 = pos2 - pos2.astype(jnp.int32).astype(jnp.float32)
                    u0 = 1.0 - t0
                    u1 = 1.0 - t1
                    u2 = 1.0 - t2
                    wyz = [u1 * u2, u1 * t2, t1 * u2, t1 * t2]
                    interp0 = jnp.zeros((LANES,), jnp.float32)
                    interp1 = jnp.zeros((LANES,), jnp.float32)
                    for k, (cx, cy, cz) in enumerate(CORNERS):
                        h = av[cx] ^ bcv[cy * 2 + cz]
                        col0 = (h & 3) << 1
                        ridx = (l * 8 + k) * C + g * LANES + lane
                        f0 = plsc.load_gather(rowsv, [ridx, col0])
                        f1 = plsc.load_gather(rowsv, [ridx, col0 + 1])
                        w = (t0 if cx else u0) * wyz[cy * 2 + cz]
                        interp0 = interp0 + f0 * w
                        interp1 = interp1 + f1 * w
                    pev[pl.ds(2 * l * C + g * LANES, LANES)] = interp0
                    pev[pl.ds((2 * l + 1) * C + g * LANES, LANES)] = interp1
                return c2

            lax.fori_loop(0, G, pass_b, 0)

            tcol = base // 128
            for t in range(3):
                pltpu.sync_copy(
                    pev.at[pl.ds(t * 8 * C, 8 * C)],
                    pe_hbm.at[pl.ds((t * NTCOL + tcol) * 1024, 1024)],
                )
            return carry

        lax.fori_loop(0, NCHUNK, chunk_body, 0)

    return functools.partial(
        pl.kernel,
        out_type=jax.ShapeDtypeStruct((IN_DIM * n_points,), jnp.float32),
        mesh=plsc.VectorSubcoreMesh(
            core_axis_name="c", subcore_axis_name="s", num_cores=NC, num_subcores=NS
        ),
        compiler_params=pltpu.CompilerParams(
            needs_layout_passes=False, use_tc_tiling_on_sc=False
        ),
        scratch_types=[
            pltpu.VMEM((C * 3,), jnp.float32),
            pltpu.VMEM((C * 8 * N_LEVELS,), jnp.int32),
            pltpu.VMEM((C * 8 * N_LEVELS, 8), jnp.float32),
            pltpu.VMEM((IN_DIM * C,), jnp.float32),
            pltpu.SemaphoreType.DMA,
        ],
        interpret=interpret,
    )(_sc_encode_body)


_sc_encode = make_sc_encode(N_POINTS)


BN = 2048
BNC = BN // 128


def _mlp_body(pe_ref, w0t_ref, b0_ref, w1t_ref, b1_ref, w2r_ref, b2r_ref, o_ref):
    w0t = w0t_ref[...]
    b0 = b0_ref[...]
    w1t = w1t_ref[...]
    b1 = b1_ref[...]
    w2r = w2r_ref[...]
    b2 = b2r_ref[...]
    zs = []
    for cb in range(BNC):
        pe_t = jnp.concatenate([pe_ref[t, cb] for t in range(3)], axis=0)
        h = jnp.maximum(
            lax.dot(w0t, pe_t, preferred_element_type=jnp.float32) + b0, 0.0
        )
        h = jnp.maximum(
            lax.dot(w1t, h, preferred_element_type=jnp.float32) + b1, 0.0
        )
        zs.append(lax.dot(w2r, h, preferred_element_type=jnp.float32) + b2)
    z = jnp.concatenate(zs, axis=0)
    o_ref[...] = jnp.maximum(z, 0.0) + jnp.log(1.0 + jnp.exp(-jnp.abs(z)))


_mlp = pl.pallas_call(
    _mlp_body,
    grid=(N_POINTS // BN,),
    in_specs=[
        pl.BlockSpec((3, BNC, 8, 128), lambda i: (0, i, 0, 0)),
        pl.BlockSpec((WIDTH, IN_DIM), lambda i: (0, 0)),
        pl.BlockSpec((WIDTH, 1), lambda i: (0, 0)),
        pl.BlockSpec((WIDTH, WIDTH), lambda i: (0, 0)),
        pl.BlockSpec((WIDTH, 1), lambda i: (0, 0)),
        pl.BlockSpec((1, WIDTH), lambda i: (0, 0)),
        pl.BlockSpec((1, 1), lambda i: (0, 0)),
    ],
    out_specs=pl.BlockSpec((BNC, 128), lambda i: (i, 0)),
    out_shape=jax.ShapeDtypeStruct((N_POINTS // 128, 128), jnp.float32),
)


def kernel(x, bounding_box, tables, W0, b0, W1, b1, W2, b2):
    xn = ((x - bounding_box[0]) / (bounding_box[1] - bounding_box[0])).T
    tbl_native = (
        tables.reshape(N_LEVELS, TABLE_SIZE // 128, 128, 2)
        .transpose(0, 1, 3, 2)
        .reshape(-1)
    )
    tbl_int = _sc_interleave(tbl_native)
    pe_flat = _sc_encode(xn.reshape(-1), tbl_int.reshape(NSUPER, 8))
    pe4 = pe_flat.reshape(3, N_POINTS // 128, 8, 128)
    dens = _mlp(
        pe4,
        W0.T,
        b0.reshape(WIDTH, 1),
        W1.T,
        b1.reshape(WIDTH, 1),
        W2[:, 0:1].T,
        b2[0:1].reshape(1, 1),
    )
    return dens.reshape(N_POINTS)


# speedup vs baseline: 6.4131x; 1.2623x over previous
import functools

import jax
import jax.numpy as jnp
import numpy as np
from jax import lax
from jax.experimental import pallas as pl
from jax.experimental.pallas import tpu as pltpu
from jax.experimental.pallas import tpu_sc as plsc

N_POINTS = 262144
N_LEVELS = 12
TABLE_SIZE = 2 ** 19
MASK = TABLE_SIZE - 1
BASE_RES = 16
LEVEL_SCALE = 1.38
IN_DIM = 2 * N_LEVELS
WIDTH = 64

RES = [float(np.floor(BASE_RES * LEVEL_SCALE ** l)) for l in range(N_LEVELS)]

P2 = int(np.uint32(2654435761).astype(np.int32))
P3 = int(np.uint32(805459861).astype(np.int32))

NC = 2
NS = 16
NW = NC * NS
LANES = 16

NTBL = N_LEVELS * TABLE_SIZE * 2
NSUPER = NTBL // 8

CORNERS = [(cx, cy, cz) for cx in (0, 1) for cy in (0, 1) for cz in (0, 1)]


ICHUNK = 16384


def _sc_interleave_body(src_hbm, dst_hbm, inv, outv):
    wid = lax.axis_index("s") * NC + lax.axis_index("c")
    per_w = NTBL // NW
    wbase = wid * per_w
    nchunk = per_w // ICHUNK
    lane = lax.iota(jnp.int32, LANES)
    perm = (lane >> 1) + (lane & 1) * 128

    def chunk(ci, carry):
        base = wbase + ci * ICHUNK
        pltpu.sync_copy(src_hbm.at[pl.ds(base, ICHUNK)], inv)

        def block(b, c2):
            boff = b * 256

            def group(gi, c3):
                vals = plsc.load_gather(inv, [boff + gi * 8 + perm])
                outv[pl.ds(boff + gi * LANES, LANES)] = vals
                return c3

            lax.fori_loop(0, 16, group, 0)
            return c2

        lax.fori_loop(0, ICHUNK // 256, block, 0)
        pltpu.sync_copy(outv, dst_hbm.at[pl.ds(base, ICHUNK)])
        return carry

    lax.fori_loop(0, nchunk, chunk, 0)


_sc_interleave = functools.partial(
    pl.kernel,
    out_type=jax.ShapeDtypeStruct((NTBL,), jnp.float32),
    mesh=plsc.VectorSubcoreMesh(
        core_axis_name="c", subcore_axis_name="s", num_cores=NC, num_subcores=NS
    ),
    compiler_params=pltpu.CompilerParams(
        needs_layout_passes=False, use_tc_tiling_on_sc=False
    ),
    scratch_types=[
        pltpu.VMEM((ICHUNK,), jnp.float32),
        pltpu.VMEM((ICHUNK,), jnp.float32),
    ],
)(_sc_interleave_body)


def make_sc_encode(n_points, interpret=False):
    C = 32
    G = C // LANES
    ROWS = C * 8 * N_LEVELS
    DMA_ROWS = 128
    NDMA = ROWS // DMA_ROWS
    PER_W = n_points // NW
    NCHUNK = PER_W // C
    NTCOL = n_points // 128

    def corner_parts(x0, x1, x2, l):
        pos0 = x0 * RES[l]
        pos1 = x1 * RES[l]
        pos2 = x2 * RES[l]
        a0 = pos0.astype(jnp.int32)
        b0 = pos1.astype(jnp.int32) * P2
        c0 = pos2.astype(jnp.int32) * P3
        parts = (
            [a0, a0 + 1],
            [b0 ^ c0, b0 ^ (c0 + P3), (b0 + P2) ^ c0, (b0 + P2) ^ (c0 + P3)],
        )
        return (pos0, pos1, pos2), parts

    def _sc_encode_body(
        xn_hbm, tbl_hbm, pe_hbm, xbig, idx0, idx1, rows0, rows1, pev, sem0, sem1
    ):
        wid = lax.axis_index("s") * NC + lax.axis_index("c")
        wbase = wid * PER_W
        pltpu.sync_copy(xn_hbm.at[pl.ds(wbase * 3, PER_W * 3)], xbig)

        lane = lax.iota(jnp.int32, LANES)
        lane3 = lane * 3

        def load_xyz(ci, g):
            off = (ci * C + g * LANES) * 3
            x0 = plsc.load_gather(xbig, [off + lane3])
            x1 = plsc.load_gather(xbig, [off + lane3 + 1])
            x2 = plsc.load_gather(xbig, [off + lane3 + 2])
            return x0, x1, x2

        def pass_a(ci, idxv):
            def body(g, c2):
                x0, x1, x2 = load_xyz(ci, g)
                for l in range(N_LEVELS):
                    _, (av, bcv) = corner_parts(x0, x1, x2, l)
                    loff = l * (TABLE_SIZE // 4)
                    for k, (cx, cy, cz) in enumerate(CORNERS):
                        h = av[cx] ^ bcv[cy * 2 + cz]
                        idxv[pl.ds((l * 8 + k) * C + g * LANES, LANES)] = (
                            loff + ((h & MASK) >> 2)
                        )
                return c2

            lax.fori_loop(0, G, body, 0)

        def fire(idxv, rowsv, sem):
            def body(j, c2):
                pltpu.async_copy(
                    tbl_hbm.at[idxv.at[pl.ds(j * DMA_ROWS, DMA_ROWS)]],
                    rowsv.at[pl.ds(j * DMA_ROWS, DMA_ROWS)],
                    sem,
                )
                return c2

            lax.fori_loop(0, NDMA, body, 0)

        def drain(idxv, rowsv, sem):
            def body(j, c2):
                pltpu.make_async_copy(
                    tbl_hbm.at[idxv.at[pl.ds(j * DMA_ROWS, DMA_ROWS)]],
                    rowsv.at[pl.ds(j * DMA_ROWS, DMA_ROWS)],
                    sem,
                ).wait()
                return c2

            lax.fori_loop(0, NDMA, body, 0)

        def pass_b(ci, rowsv):
            pcol = (ci & 3) * C

            def body(g, c2):
                x0, x1, x2 = load_xyz(ci, g)
                for l in range(N_LEVELS):
                    (pos0, pos1, pos2), (av, bcv) = corner_parts(x0, x1, x2, l)
                    t0 = pos0 - pos0.astype(jnp.int32).astype(jnp.float32)
                    t1 = pos1 - pos1.astype(jnp.int32).astype(jnp.float32)
                    t2 = pos2 - pos2.astype(jnp.int32).astype(jnp.float32)
                    u0 = 1.0 - t0
                    u1 = 1.0 - t1
                    u2 = 1.0 - t2
                    wyz = [u1 * u2, u1 * t2, t1 * u2, t1 * t2]
                    interp0 = jnp.zeros((LANES,), jnp.float32)
                    interp1 = jnp.zeros((LANES,), jnp.float32)
                    for k, (cx, cy, cz) in enumerate(CORNERS):
                        h = av[cx] ^ bcv[cy * 2 + cz]
                        col0 = (h & 3) << 1
                        ridx = (l * 8 + k) * C + g * LANES + lane
                        f0 = plsc.load_gather(rowsv, [ridx, col0])
                        f1 = plsc.load_gather(rowsv, [ridx, col0 + 1])
                        w = (t0 if cx else u0) * wyz[cy * 2 + cz]
                        interp0 = interp0 + f0 * w
                        interp1 = interp1 + f1 * w
                    pev[pl.ds(2 * l * 128 + pcol + g * LANES, LANES)] = interp0
                    pev[pl.ds((2 * l + 1) * 128 + pcol + g * LANES, LANES)] = (
                        interp1
                    )
                return c2

            lax.fori_loop(0, G, body, 0)

            @pl.when((ci & 3) == 3)
            def _():
                tcol = (wbase + (ci - 3) * C) // 128
                for t in range(3):
                    pltpu.sync_copy(
                        pev.at[pl.ds(t * 8 * 128, 8 * 128)],
                        pe_hbm.at[pl.ds((t * NTCOL + tcol) * 1024, 1024)],
                    )

        pass_a(0, idx0)
        fire(idx0, rows0, sem0)

        def pipe(i2, carry):
            even = 2 * i2
            odd = even + 1
            pass_a(odd, idx1)
            fire(idx1, rows1, sem1)
            drain(idx0, rows0, sem0)
            pass_b(even, rows0)

            @pl.when(i2 + 1 < NCHUNK // 2)
            def _():
                pass_a(even + 2, idx0)
                fire(idx0, rows0, sem0)

            drain(idx1, rows1, sem1)
            pass_b(odd, rows1)
            return carry

        lax.fori_loop(0, NCHUNK // 2, pipe, 0)

    return functools.partial(
        pl.kernel,
        out_type=jax.ShapeDtypeStruct((IN_DIM * n_points,), jnp.float32),
        mesh=plsc.VectorSubcoreMesh(
            core_axis_name="c", subcore_axis_name="s", num_cores=NC, num_subcores=NS
        ),
        compiler_params=pltpu.CompilerParams(
            needs_layout_passes=False, use_tc_tiling_on_sc=False
        ),
        scratch_types=[
            pltpu.VMEM((PER_W * 3,), jnp.float32),
            pltpu.VMEM((ROWS,), jnp.int32),
            pltpu.VMEM((ROWS,), jnp.int32),
            pltpu.VMEM((ROWS, 8), jnp.float32),
            pltpu.VMEM((ROWS, 8), jnp.float32),
            pltpu.VMEM((IN_DIM * 128,), jnp.float32),
            pltpu.SemaphoreType.DMA,
            pltpu.SemaphoreType.DMA,
        ],
        interpret=interpret,
    )(_sc_encode_body)


_sc_encode = make_sc_encode(N_POINTS)


BN = 2048
BNC = BN // 128


def _mlp_body(pe_ref, w0t_ref, b0_ref, w1t_ref, b1_ref, w2r_ref, b2r_ref, o_ref):
    w0t = w0t_ref[...]
    b0 = b0_ref[...]
    w1t = w1t_ref[...]
    b1 = b1_ref[...]
    w2r = w2r_ref[...]
    b2 = b2r_ref[...]
    zs = []
    for cb in range(BNC):
        pe_t = jnp.concatenate([pe_ref[t, cb] for t in range(3)], axis=0)
        h = jnp.maximum(
            lax.dot(w0t, pe_t, preferred_element_type=jnp.float32) + b0, 0.0
        )
        h = jnp.maximum(
            lax.dot(w1t, h, preferred_element_type=jnp.float32) + b1, 0.0
        )
        zs.append(lax.dot(w2r, h, preferred_element_type=jnp.float32) + b2)
    z = jnp.concatenate(zs, axis=0)
    o_ref[...] = jnp.maximum(z, 0.0) + jnp.log(1.0 + jnp.exp(-jnp.abs(z)))


_mlp = pl.pallas_call(
    _mlp_body,
    grid=(N_POINTS // BN,),
    in_specs=[
        pl.BlockSpec((3, BNC, 8, 128), lambda i: (0, i, 0, 0)),
        pl.BlockSpec((WIDTH, IN_DIM), lambda i: (0, 0)),
        pl.BlockSpec((WIDTH, 1), lambda i: (0, 0)),
        pl.BlockSpec((WIDTH, WIDTH), lambda i: (0, 0)),
        pl.BlockSpec((WIDTH, 1), lambda i: (0, 0)),
        pl.BlockSpec((1, WIDTH), lambda i: (0, 0)),
        pl.BlockSpec((1, 1), lambda i: (0, 0)),
    ],
    out_specs=pl.BlockSpec((BNC, 128), lambda i: (i, 0)),
    out_shape=jax.ShapeDtypeStruct((N_POINTS // 128, 128), jnp.float32),
)


def kernel(x, bounding_box, tables, W0, b0, W1, b1, W2, b2):
    xn = (x - bounding_box[0]) / (bounding_box[1] - bounding_box[0])
    tbl_native = (
        tables.reshape(N_LEVELS, TABLE_SIZE // 128, 128, 2)
        .transpose(0, 1, 3, 2)
        .reshape(-1)
    )
    tbl_int = _sc_interleave(tbl_native)
    pe_flat = _sc_encode(xn.reshape(-1), tbl_int.reshape(NSUPER, 8))
    pe4 = pe_flat.reshape(3, N_POINTS // 128, 8, 128)
    dens = _mlp(
        pe4,
        W0.T,
        b0.reshape(WIDTH, 1),
        W1.T,
        b1.reshape(WIDTH, 1),
        W2[:, 0:1].T,
        b2[0:1].reshape(1, 1),
    )
    return dens.reshape(N_POINTS)


# speedup vs baseline: 7.9300x; 1.2365x over previous
import functools

import jax
import jax.numpy as jnp
import numpy as np
from jax import lax
from jax.experimental import pallas as pl
from jax.experimental.pallas import tpu as pltpu
from jax.experimental.pallas import tpu_sc as plsc

N_POINTS = 262144
N_LEVELS = 12
TABLE_SIZE = 2 ** 19
MASK = TABLE_SIZE - 1
BASE_RES = 16
LEVEL_SCALE = 1.38
IN_DIM = 2 * N_LEVELS
WIDTH = 64

RES = [float(np.floor(BASE_RES * LEVEL_SCALE ** l)) for l in range(N_LEVELS)]

P2 = int(np.uint32(2654435761).astype(np.int32))
P3 = int(np.uint32(805459861).astype(np.int32))

NC = 2
NS = 16
NW = NC * NS
LANES = 16

NTBL = N_LEVELS * TABLE_SIZE * 2
NSUPER = NTBL // 8

CORNERS = [(cx, cy, cz) for cx in (0, 1) for cy in (0, 1) for cz in (0, 1)]


ICHUNK = 16384


def _sc_interleave_body(src_hbm, dst_hbm, inv, outv):
    wid = lax.axis_index("s") * NC + lax.axis_index("c")
    per_w = NTBL // NW
    wbase = wid * per_w
    nchunk = per_w // ICHUNK
    lane = lax.iota(jnp.int32, LANES)
    perm = (lane >> 1) + (lane & 1) * 128

    def chunk(ci, carry):
        base = wbase + ci * ICHUNK
        pltpu.sync_copy(src_hbm.at[pl.ds(base, ICHUNK)], inv)

        def block(b, c2):
            boff = b * 256

            def group(gi, c3):
                vals = plsc.load_gather(inv, [boff + gi * 8 + perm])
                outv[pl.ds(boff + gi * LANES, LANES)] = vals
                return c3

            lax.fori_loop(0, 16, group, 0)
            return c2

        lax.fori_loop(0, ICHUNK // 256, block, 0)
        pltpu.sync_copy(outv, dst_hbm.at[pl.ds(base, ICHUNK)])
        return carry

    lax.fori_loop(0, nchunk, chunk, 0)


_sc_interleave = functools.partial(
    pl.kernel,
    out_type=jax.ShapeDtypeStruct((NTBL,), jnp.float32),
    mesh=plsc.VectorSubcoreMesh(
        core_axis_name="c", subcore_axis_name="s", num_cores=NC, num_subcores=NS
    ),
    compiler_params=pltpu.CompilerParams(
        needs_layout_passes=False, use_tc_tiling_on_sc=False
    ),
    scratch_types=[
        pltpu.VMEM((ICHUNK,), jnp.float32),
        pltpu.VMEM((ICHUNK,), jnp.float32),
    ],
)(_sc_interleave_body)


def make_sc_encode(n_points, interpret=False):
    C = 32
    G = C // LANES
    ROWS = C * 8 * N_LEVELS
    DMA_ROWS = 128
    NDMA = ROWS // DMA_ROWS
    PER_W = n_points // NW
    NCHUNK = PER_W // C
    NTCOL = n_points // 128

    def corner_parts(x0, x1, x2, l):
        pos0 = x0 * RES[l]
        pos1 = x1 * RES[l]
        pos2 = x2 * RES[l]
        a0 = pos0.astype(jnp.int32)
        b0 = pos1.astype(jnp.int32) * P2
        c0 = pos2.astype(jnp.int32) * P3
        parts = (
            [a0, a0 + 1],
            [b0 ^ c0, b0 ^ (c0 + P3), (b0 + P2) ^ c0, (b0 + P2) ^ (c0 + P3)],
        )
        return (pos0, pos1, pos2), parts

    def _sc_encode_body(
        xn_hbm, tbl_hbm, pe_hbm, xbig, idx0, idx1, rows0, rows1, pev, sem0, sem1
    ):
        wid = lax.axis_index("s") * NC + lax.axis_index("c")
        wbase = wid * PER_W
        pltpu.sync_copy(xn_hbm.at[pl.ds(wbase * 3, PER_W * 3)], xbig)

        lane = lax.iota(jnp.int32, LANES)
        lane3 = lane * 3

        def load_xyz(ci, g):
            off = (ci * C + g * LANES) * 3
            x0 = plsc.load_gather(xbig, [off + lane3])
            x1 = plsc.load_gather(xbig, [off + lane3 + 1])
            x2 = plsc.load_gather(xbig, [off + lane3 + 2])
            return x0, x1, x2

        def pass_a(ci, idxv):
            def body(g, c2):
                x0, x1, x2 = load_xyz(ci, g)
                for l in range(N_LEVELS):
                    _, (av, bcv) = corner_parts(x0, x1, x2, l)
                    loff = l * (TABLE_SIZE // 4)
                    for k, (cx, cy, cz) in enumerate(CORNERS):
                        h = av[cx] ^ bcv[cy * 2 + cz]
                        idxv[pl.ds((l * 8 + k) * C + g * LANES, LANES)] = (
                            loff + ((h & MASK) >> 2)
                        )
                return c2

            lax.fori_loop(0, G, body, 0)

        def fire(idxv, rowsv, sem):
            def body(j, c2):
                pltpu.async_copy(
                    tbl_hbm.at[idxv.at[pl.ds(j * DMA_ROWS, DMA_ROWS)]],
                    rowsv.at[pl.ds(j * DMA_ROWS, DMA_ROWS)],
                    sem,
                )
                return c2

            lax.fori_loop(0, NDMA, body, 0)

        def drain(idxv, rowsv, sem):
            def body(j, c2):
                pltpu.make_async_copy(
                    tbl_hbm.at[idxv.at[pl.ds(j * DMA_ROWS, DMA_ROWS)]],
                    rowsv.at[pl.ds(j * DMA_ROWS, DMA_ROWS)],
                    sem,
                ).wait()
                return c2

            lax.fori_loop(0, NDMA, body, 0)

        def pass_b(ci, rowsv):
            pcol = (ci & 3) * C

            def body(g, c2):
                x0, x1, x2 = load_xyz(ci, g)
                for l in range(N_LEVELS):
                    (pos0, pos1, pos2), (av, bcv) = corner_parts(x0, x1, x2, l)
                    t0 = pos0 - pos0.astype(jnp.int32).astype(jnp.float32)
                    t1 = pos1 - pos1.astype(jnp.int32).astype(jnp.float32)
                    t2 = pos2 - pos2.astype(jnp.int32).astype(jnp.float32)
                    u0 = 1.0 - t0
                    u1 = 1.0 - t1
                    u2 = 1.0 - t2
                    wyz = [u1 * u2, u1 * t2, t1 * u2, t1 * t2]
                    interp0 = jnp.zeros((LANES,), jnp.float32)
                    interp1 = jnp.zeros((LANES,), jnp.float32)
                    for k, (cx, cy, cz) in enumerate(CORNERS):
                        h = av[cx] ^ bcv[cy * 2 + cz]
                        col0 = (h & 3) << 1
                        ridx = (l * 8 + k) * C + g * LANES + lane
                        f0 = plsc.load_gather(rowsv, [ridx, col0])
                        f1 = plsc.load_gather(rowsv, [ridx, col0 + 1])
                        w = (t0 if cx else u0) * wyz[cy * 2 + cz]
                        interp0 = interp0 + f0 * w
                        interp1 = interp1 + f1 * w
                    pev[pl.ds(2 * l * 128 + pcol + g * LANES, LANES)] = interp0
                    pev[pl.ds((2 * l + 1) * 128 + pcol + g * LANES, LANES)] = (
                        interp1
                    )
                return c2

            lax.fori_loop(0, G, body, 0)

            @pl.when((ci & 3) == 3)
            def _():
                tcol = (wbase + (ci - 3) * C) // 128
                for t in range(3):
                    pltpu.sync_copy(
                        pev.at[pl.ds(t * 8 * 128, 8 * 128)],
                        pe_hbm.at[pl.ds((t * NTCOL + tcol) * 1024, 1024)],
                    )

        pass_a(0, idx0)
        fire(idx0, rows0, sem0)

        def pipe(i2, carry):
            even = 2 * i2
            odd = even + 1
            pass_a(odd, idx1)
            fire(idx1, rows1, sem1)
            drain(idx0, rows0, sem0)
            pass_b(even, rows0)

            @pl.when(i2 + 1 < NCHUNK // 2)
            def _():
                pass_a(even + 2, idx0)
                fire(idx0, rows0, sem0)

            drain(idx1, rows1, sem1)
            pass_b(odd, rows1)
            return carry

        lax.fori_loop(0, NCHUNK // 2, pipe, 0)

    return functools.partial(
        pl.kernel,
        out_type=jax.ShapeDtypeStruct((IN_DIM * n_points,), jnp.float32),
        mesh=plsc.VectorSubcoreMesh(
            core_axis_name="c", subcore_axis_name="s", num_cores=NC, num_subcores=NS
        ),
        compiler_params=pltpu.CompilerParams(
            needs_layout_passes=False, use_tc_tiling_on_sc=False
        ),
        scratch_types=[
            pltpu.VMEM((PER_W * 3,), jnp.float32),
            pltpu.VMEM((ROWS,), jnp.int32),
            pltpu.VMEM((ROWS,), jnp.int32),
            pltpu.VMEM((ROWS, 8), jnp.float32),
            pltpu.VMEM((ROWS, 8), jnp.float32),
            pltpu.VMEM((IN_DIM * 128,), jnp.float32),
            pltpu.SemaphoreType.DMA,
            pltpu.SemaphoreType.DMA,
        ],
        interpret=interpret,
    )(_sc_encode_body)


_sc_encode = make_sc_encode(N_POINTS)


BN = 2048
BNC = BN // 128


def _mlp_body(pe_ref, w0t_ref, b0_ref, w1t_ref, b1_ref, w2r_ref, b2r_ref, o_ref):
    b0 = b0_ref[...]
    b1 = b1_ref[...]
    b2 = b2r_ref[...]
    s = None
    for t in range(3):
        pe_t = jnp.concatenate([pe_ref[t, cb] for cb in range(BNC)], axis=1)
        d = lax.dot(
            w0t_ref[:, t * 8 : (t + 1) * 8],
            pe_t,
            preferred_element_type=jnp.float32,
        )
        s = d if s is None else s + d
    h = jnp.maximum(s + b0, 0.0)
    h = jnp.maximum(
        lax.dot(w1t_ref[...], h, preferred_element_type=jnp.float32) + b1, 0.0
    )
    z = lax.dot(w2r_ref[...], h, preferred_element_type=jnp.float32) + b2
    o_ref[...] = jnp.maximum(z, 0.0) + jnp.log(1.0 + jnp.exp(-jnp.abs(z)))


_mlp = pl.pallas_call(
    _mlp_body,
    grid=(N_POINTS // BN,),
    in_specs=[
        pl.BlockSpec((3, BNC, 8, 128), lambda i: (0, i, 0, 0)),
        pl.BlockSpec((WIDTH, IN_DIM), lambda i: (0, 0)),
        pl.BlockSpec((WIDTH, 1), lambda i: (0, 0)),
        pl.BlockSpec((WIDTH, WIDTH), lambda i: (0, 0)),
        pl.BlockSpec((WIDTH, 1), lambda i: (0, 0)),
        pl.BlockSpec((1, WIDTH), lambda i: (0, 0)),
        pl.BlockSpec((1, 1), lambda i: (0, 0)),
    ],
    out_specs=pl.BlockSpec((1, BN), lambda i: (0, i)),
    out_shape=jax.ShapeDtypeStruct((1, N_POINTS), jnp.float32),
)


def kernel(x, bounding_box, tables, W0, b0, W1, b1, W2, b2):
    xn = (x - bounding_box[0]) / (bounding_box[1] - bounding_box[0])
    tbl_native = (
        tables.reshape(N_LEVELS, TABLE_SIZE // 128, 128, 2)
        .transpose(0, 1, 3, 2)
        .reshape(-1)
    )
    tbl_int = _sc_interleave(tbl_native)
    pe_flat = _sc_encode(xn.reshape(-1), tbl_int.reshape(NSUPER, 8))
    pe4 = pe_flat.reshape(3, N_POINTS // 128, 8, 128)
    dens = _mlp(
        pe4,
        W0.T,
        b0.reshape(WIDTH, 1),
        W1.T,
        b1.reshape(WIDTH, 1),
        W2[:, 0:1].T,
        b2[0:1].reshape(1, 1),
    )
    return dens.reshape(N_POINTS)


# speedup vs baseline: 7.9531x; 1.0029x over previous
import functools

import jax
import jax.numpy as jnp
import numpy as np
from jax import lax
from jax.experimental import pallas as pl
from jax.experimental.pallas import tpu as pltpu
from jax.experimental.pallas import tpu_sc as plsc

N_POINTS = 262144
N_LEVELS = 12
TABLE_SIZE = 2 ** 19
MASK = TABLE_SIZE - 1
BASE_RES = 16
LEVEL_SCALE = 1.38
IN_DIM = 2 * N_LEVELS
WIDTH = 64

RES = [float(np.floor(BASE_RES * LEVEL_SCALE ** l)) for l in range(N_LEVELS)]

P2 = int(np.uint32(2654435761).astype(np.int32))
P3 = int(np.uint32(805459861).astype(np.int32))

NC = 2
NS = 16
NW = NC * NS
LANES = 16

NTBL = N_LEVELS * TABLE_SIZE * 2
NSUPER = NTBL // 8

CORNERS = [(cx, cy, cz) for cx in (0, 1) for cy in (0, 1) for cz in (0, 1)]


ICHUNK = 16384


def _sc_interleave_body(src_hbm, dst_hbm, inv, outv):
    wid = lax.axis_index("s") * NC + lax.axis_index("c")
    per_w = NTBL // NW
    wbase = wid * per_w
    nchunk = per_w // ICHUNK
    lane = lax.iota(jnp.int32, LANES)
    perm = (lane >> 1) + (lane & 1) * 128

    def chunk(ci, carry):
        base = wbase + ci * ICHUNK
        pltpu.sync_copy(src_hbm.at[pl.ds(base, ICHUNK)], inv)

        def block(b, c2):
            boff = b * 256

            def group(gi, c3):
                vals = plsc.load_gather(inv, [boff + gi * 8 + perm])
                outv[pl.ds(boff + gi * LANES, LANES)] = vals
                return c3

            lax.fori_loop(0, 16, group, 0)
            return c2

        lax.fori_loop(0, ICHUNK // 256, block, 0)
        pltpu.sync_copy(outv, dst_hbm.at[pl.ds(base, ICHUNK)])
        return carry

    lax.fori_loop(0, nchunk, chunk, 0)


_sc_interleave = functools.partial(
    pl.kernel,
    out_type=jax.ShapeDtypeStruct((NTBL,), jnp.float32),
    mesh=plsc.VectorSubcoreMesh(
        core_axis_name="c", subcore_axis_name="s", num_cores=NC, num_subcores=NS
    ),
    compiler_params=pltpu.CompilerParams(
        needs_layout_passes=False, use_tc_tiling_on_sc=False
    ),
    scratch_types=[
        pltpu.VMEM((ICHUNK,), jnp.float32),
        pltpu.VMEM((ICHUNK,), jnp.float32),
    ],
)(_sc_interleave_body)


def make_sc_encode(n_points, interpret=False):
    C = 32
    G = C // LANES
    ROWS = C * 8 * N_LEVELS
    DMA_ROWS = 128
    NDMA = ROWS // DMA_ROWS
    PER_W = n_points // NW
    NCHUNK = PER_W // C
    NTCOL = n_points // 128

    def corner_parts(x0, x1, x2, l):
        pos0 = x0 * RES[l]
        pos1 = x1 * RES[l]
        pos2 = x2 * RES[l]
        a0 = pos0.astype(jnp.int32)
        b0 = pos1.astype(jnp.int32) * P2
        c0 = pos2.astype(jnp.int32) * P3
        parts = (
            [a0, a0 + 1],
            [b0 ^ c0, b0 ^ (c0 + P3), (b0 + P2) ^ c0, (b0 + P2) ^ (c0 + P3)],
        )
        return (pos0, pos1, pos2), parts

    def _sc_encode_body(
        xn_hbm, tbl_hbm, pe_hbm, xbig, idx0, idx1, rows0, rows1, pev, sem0, sem1
    ):
        wid = lax.axis_index("s") * NC + lax.axis_index("c")
        wbase = wid * PER_W
        pltpu.sync_copy(xn_hbm.at[pl.ds(wbase * 3, PER_W * 3)], xbig)

        lane = lax.iota(jnp.int32, LANES)
        lane3 = lane * 3

        def load_xyz(ci, g):
            off = (ci * C + g * LANES) * 3
            x0 = plsc.load_gather(xbig, [off + lane3])
            x1 = plsc.load_gather(xbig, [off + lane3 + 1])
            x2 = plsc.load_gather(xbig, [off + lane3 + 2])
            return x0, x1, x2

        def pass_a(ci, idxv):
            def body(g, c2):
                x0, x1, x2 = load_xyz(ci, g)
                for l in range(N_LEVELS):
                    _, (av, bcv) = corner_parts(x0, x1, x2, l)
                    loff = l * (TABLE_SIZE // 4)
                    for k, (cx, cy, cz) in enumerate(CORNERS):
                        h = av[cx] ^ bcv[cy * 2 + cz]
                        idxv[pl.ds((l * 8 + k) * C + g * LANES, LANES)] = (
                            loff + ((h & MASK) >> 2)
                        )
                return c2

            lax.fori_loop(0, G, body, 0)

        def fire(idxv, rowsv, sem):
            def body(j, c2):
                pltpu.async_copy(
                    tbl_hbm.at[idxv.at[pl.ds(j * DMA_ROWS, DMA_ROWS)]],
                    rowsv.at[pl.ds(j * DMA_ROWS, DMA_ROWS)],
                    sem,
                )
                return c2

            lax.fori_loop(0, NDMA, body, 0)

        def drain(idxv, rowsv, sem):
            pltpu.make_async_copy(tbl_hbm.at[idxv], rowsv, sem).wait()

        def pass_b(ci, rowsv):
            pcol = (ci & 3) * C

            def body(g, c2):
                x0, x1, x2 = load_xyz(ci, g)
                for l in range(N_LEVELS):
                    (pos0, pos1, pos2), (av, bcv) = corner_parts(x0, x1, x2, l)
                    t0 = pos0 - pos0.astype(jnp.int32).astype(jnp.float32)
                    t1 = pos1 - pos1.astype(jnp.int32).astype(jnp.float32)
                    t2 = pos2 - pos2.astype(jnp.int32).astype(jnp.float32)
                    u0 = 1.0 - t0
                    u1 = 1.0 - t1
                    u2 = 1.0 - t2
                    wyz = [u1 * u2, u1 * t2, t1 * u2, t1 * t2]
                    interp0 = jnp.zeros((LANES,), jnp.float32)
                    interp1 = jnp.zeros((LANES,), jnp.float32)
                    for k, (cx, cy, cz) in enumerate(CORNERS):
                        h = av[cx] ^ bcv[cy * 2 + cz]
                        col0 = (h & 3) << 1
                        ridx = (l * 8 + k) * C + g * LANES + lane
                        f0 = plsc.load_gather(rowsv, [ridx, col0])
                        f1 = plsc.load_gather(rowsv, [ridx, col0 + 1])
                        w = (t0 if cx else u0) * wyz[cy * 2 + cz]
                        interp0 = interp0 + f0 * w
                        interp1 = interp1 + f1 * w
                    pev[pl.ds(2 * l * 128 + pcol + g * LANES, LANES)] = interp0
                    pev[pl.ds((2 * l + 1) * 128 + pcol + g * LANES, LANES)] = (
                        interp1
                    )
                return c2

            lax.fori_loop(0, G, body, 0)

            @pl.when((ci & 3) == 3)
            def _():
                tcol = (wbase + (ci - 3) * C) // 128
                for t in range(3):
                    pltpu.sync_copy(
                        pev.at[pl.ds(t * 8 * 128, 8 * 128)],
                        pe_hbm.at[pl.ds((t * NTCOL + tcol) * 1024, 1024)],
                    )

        pass_a(0, idx0)
        fire(idx0, rows0, sem0)

        def pipe(i2, carry):
            even = 2 * i2
            odd = even + 1
            pass_a(odd, idx1)
            fire(idx1, rows1, sem1)
            drain(idx0, rows0, sem0)
            pass_b(even, rows0)

            @pl.when(i2 + 1 < NCHUNK // 2)
            def _():
                pass_a(even + 2, idx0)
                fire(idx0, rows0, sem0)

            drain(idx1, rows1, sem1)
            pass_b(odd, rows1)
            return carry

        lax.fori_loop(0, NCHUNK // 2, pipe, 0)

    return functools.partial(
        pl.kernel,
        out_type=jax.ShapeDtypeStruct((IN_DIM * n_points,), jnp.float32),
        mesh=plsc.VectorSubcoreMesh(
            core_axis_name="c", subcore_axis_name="s", num_cores=NC, num_subcores=NS
        ),
        compiler_params=pltpu.CompilerParams(
            needs_layout_passes=False, use_tc_tiling_on_sc=False
        ),
        scratch_types=[
            pltpu.VMEM((PER_W * 3,), jnp.float32),
            pltpu.VMEM((ROWS,), jnp.int32),
            pltpu.VMEM((ROWS,), jnp.int32),
            pltpu.VMEM((ROWS, 8), jnp.float32),
            pltpu.VMEM((ROWS, 8), jnp.float32),
            pltpu.VMEM((IN_DIM * 128,), jnp.float32),
            pltpu.SemaphoreType.DMA,
            pltpu.SemaphoreType.DMA,
        ],
        interpret=interpret,
    )(_sc_encode_body)


_sc_encode = make_sc_encode(N_POINTS)


BN = 2048
BNC = BN // 128


def _mlp_body(pe_ref, w0t_ref, b0_ref, w1t_ref, b1_ref, w2r_ref, b2r_ref, o_ref):
    b0 = b0_ref[...]
    b1 = b1_ref[...]
    b2 = b2r_ref[...]
    s = None
    for t in range(3):
        pe_t = jnp.concatenate([pe_ref[t, cb] for cb in range(BNC)], axis=1)
        d = lax.dot(
            w0t_ref[:, t * 8 : (t + 1) * 8],
            pe_t,
            preferred_element_type=jnp.float32,
        )
        s = d if s is None else s + d
    h = jnp.maximum(s + b0, 0.0)
    h = jnp.maximum(
        lax.dot(w1t_ref[...], h, preferred_element_type=jnp.float32) + b1, 0.0
    )
    z = lax.dot(w2r_ref[...], h, preferred_element_type=jnp.float32) + b2
    o_ref[...] = jnp.maximum(z, 0.0) + jnp.log(1.0 + jnp.exp(-jnp.abs(z)))


_mlp = pl.pallas_call(
    _mlp_body,
    grid=(N_POINTS // BN,),
    in_specs=[
        pl.BlockSpec((3, BNC, 8, 128), lambda i: (0, i, 0, 0)),
        pl.BlockSpec((WIDTH, IN_DIM), lambda i: (0, 0)),
        pl.BlockSpec((WIDTH, 1), lambda i: (0, 0)),
        pl.BlockSpec((WIDTH, WIDTH), lambda i: (0, 0)),
        pl.BlockSpec((WIDTH, 1), lambda i: (0, 0)),
        pl.BlockSpec((1, WIDTH), lambda i: (0, 0)),
        pl.BlockSpec((1, 1), lambda i: (0, 0)),
    ],
    out_specs=pl.BlockSpec((1, BN), lambda i: (0, i)),
    out_shape=jax.ShapeDtypeStruct((1, N_POINTS), jnp.float32),
)


def kernel(x, bounding_box, tables, W0, b0, W1, b1, W2, b2):
    xn = (x - bounding_box[0]) / (bounding_box[1] - bounding_box[0])
    tbl_native = (
        tables.reshape(N_LEVELS, TABLE_SIZE // 128, 128, 2)
        .transpose(0, 1, 3, 2)
        .reshape(-1)
    )
    tbl_int = _sc_interleave(tbl_native)
    pe_flat = _sc_encode(xn.reshape(-1), tbl_int.reshape(NSUPER, 8))
    pe4 = pe_flat.reshape(3, N_POINTS // 128, 8, 128)
    dens = _mlp(
        pe4,
        W0.T,
        b0.reshape(WIDTH, 1),
        W1.T,
        b1.reshape(WIDTH, 1),
        W2[:, 0:1].T,
        b2[0:1].reshape(1, 1),
    )
    return dens.reshape(N_POINTS)
